# baseline (speedup 1.0000x reference)
import functools

import jax
import jax.numpy as jnp
from jax import lax
from jax.experimental import pallas as pl
from jax.experimental.pallas import tpu as pltpu
from jax.experimental.pallas import tpu_sc as plsc

D = 64
NFIELD = 3
B = 16384
V = 100000
ROW_W = NFIELD * D
IDXH = B // 2
IDXQ = B // 4

_info = plsc.get_sparse_core_info()
_NC, _NS = _info.num_cores, _info.num_subcores
NW = _NC * _NS
DPW = D // NW
L = 16


def _body(
    uid, iid, cid, wtu, wti, wtc, outT,
    idx0, idx1, row_v, col_v, sem_r, sem_w, sem_i,
):
    wid = lax.axis_index("s") * _NC + lax.axis_index("c")
    ibufs = (idx0, idx1)

    wcps = {}
    for f, (idx_hbm, wt) in enumerate(((uid, wtu), (iid, wti), (cid, wtc))):
        for i in range(DPW):
            d = wid * DPW + i
            c = f * D + d
            rcp = pltpu.async_copy(wt.at[pl.ds(d, 1)], row_v, sem_r)
            icp = pltpu.async_copy(idx_hbm.at[pl.ds(0, IDXQ)], ibufs[0], sem_i)
            for q in range(4):
                nicp = None
                if q < 3:
                    nicp = pltpu.async_copy(
                        idx_hbm.at[pl.ds((q + 1) * IDXQ, IDXQ)],
                        ibufs[(q + 1) % 2],
                        sem_i,
                    )
                icp.wait()
                if q == 0:
                    rcp.wait()
                if q % 2 == 0 and (q // 2) in wcps:
                    wcps[q // 2].wait()
                ib = ibufs[q % 2]

                @plsc.parallel_loop(0, IDXQ, step=L, unroll=8)
                def _g16(j, ib=ib, q=q):
                    vi = ib[pl.ds(j, L)]
                    vals = plsc.load_gather(row_v.at[0], [vi])
                    col_v[0, pl.ds(q * IDXQ + j, L)] = vals
                if q % 2 == 1:
                    h = q // 2
                    wcps[h] = pltpu.async_copy(
                        col_v.at[pl.ds(0, 1), pl.ds(h * IDXH, IDXH)],
                        outT.at[pl.ds(c, 1), pl.ds(h * IDXH, IDXH)],
                        sem_w,
                    )
                icp = nicp
    for h in wcps:
        wcps[h].wait()


@jax.jit
def kernel(user_id, item_id, cat_id, W_user, W_item, W_cat):
    mesh = plsc.VectorSubcoreMesh(core_axis_name="c", subcore_axis_name="s")
    run = functools.partial(
        pl.kernel,
        out_type=jax.ShapeDtypeStruct((ROW_W, B), jnp.float32),
        scratch_types=[
            pltpu.VMEM((IDXQ,), jnp.int32),
            pltpu.VMEM((IDXQ,), jnp.int32),
            pltpu.VMEM((1, V), jnp.float32),
            pltpu.VMEM((1, B), jnp.float32),
            pltpu.SemaphoreType.DMA,
            pltpu.SemaphoreType.DMA,
            pltpu.SemaphoreType.DMA,
        ],
        mesh=mesh,
        compiler_params=pltpu.CompilerParams(needs_layout_passes=False),
    )(_body)
    outT = run(
        user_id.astype(jnp.int32),
        item_id.astype(jnp.int32),
        cat_id.astype(jnp.int32),
        W_user.T,
        W_item.T,
        W_cat.T,
    )
    return outT.T


# speedup vs baseline: 1.0033x; 1.0033x over previous
import functools

import jax
import jax.numpy as jnp
from jax import lax
from jax.experimental import pallas as pl
from jax.experimental.pallas import tpu as pltpu
from jax.experimental.pallas import tpu_sc as plsc

D = 64
NFIELD = 3
B = 16384
V = 100000
ROW_W = NFIELD * D
IDXH = B // 2
IDXQ = B // 4

_info = plsc.get_sparse_core_info()
_NC, _NS = _info.num_cores, _info.num_subcores
NW = _NC * _NS
DPW = D // NW
L = 16


def _body(
    uid, iid, cid, wtu, wti, wtc, outT,
    idx0, idx1, row_v, col_v, sem_r, sem_w0, sem_w1, sem_i0, sem_i1,
):
    wid = lax.axis_index("s") * _NC + lax.axis_index("c")
    ibufs = (idx0, idx1)
    isems = (sem_i0, sem_i1)
    wsems = (sem_w0, sem_w1)

    wcps = {}
    for f, (idx_hbm, wt) in enumerate(((uid, wtu), (iid, wti), (cid, wtc))):
        for i in range(DPW):
            d = wid * DPW + i
            c = f * D + d
            rcp = pltpu.async_copy(wt.at[pl.ds(d, 1)], row_v, sem_r)
            icp = pltpu.async_copy(idx_hbm.at[pl.ds(0, IDXQ)], ibufs[0], isems[0])
            for q in range(4):
                nicp = None
                if q < 3:
                    nicp = pltpu.async_copy(
                        idx_hbm.at[pl.ds((q + 1) * IDXQ, IDXQ)],
                        ibufs[(q + 1) % 2],
                        isems[(q + 1) % 2],
                    )
                icp.wait()
                if q == 0:
                    rcp.wait()
                if q % 2 == 0 and (q // 2) in wcps:
                    wcps[q // 2].wait()
                ib = ibufs[q % 2]

                @plsc.parallel_loop(0, IDXQ, step=L, unroll=4)
                def _g16(j, ib=ib, q=q):
                    vi = ib[pl.ds(j, L)]
                    vals = plsc.load_gather(row_v.at[0], [vi])
                    col_v[0, pl.ds(q * IDXQ + j, L)] = vals
                if q % 2 == 1:
                    h = q // 2
                    wcps[h] = pltpu.async_copy(
                        col_v.at[pl.ds(0, 1), pl.ds(h * IDXH, IDXH)],
                        outT.at[pl.ds(c, 1), pl.ds(h * IDXH, IDXH)],
                        wsems[h],
                    )
                icp = nicp
    for h in wcps:
        wcps[h].wait()


@jax.jit
def kernel(user_id, item_id, cat_id, W_user, W_item, W_cat):
    mesh = plsc.VectorSubcoreMesh(core_axis_name="c", subcore_axis_name="s")
    run = functools.partial(
        pl.kernel,
        out_type=jax.ShapeDtypeStruct((ROW_W, B), jnp.float32),
        scratch_types=[
            pltpu.VMEM((IDXQ,), jnp.int32),
            pltpu.VMEM((IDXQ,), jnp.int32),
            pltpu.VMEM((1, V), jnp.float32),
            pltpu.VMEM((1, B), jnp.float32),
            pltpu.SemaphoreType.DMA,
            pltpu.SemaphoreType.DMA,
            pltpu.SemaphoreType.DMA,
            pltpu.SemaphoreType.DMA,
            pltpu.SemaphoreType.DMA,
        ],
        mesh=mesh,
        compiler_params=pltpu.CompilerParams(needs_layout_passes=False),
    )(_body)
    outT = run(
        user_id.astype(jnp.int32),
        item_id.astype(jnp.int32),
        cat_id.astype(jnp.int32),
        W_user.T,
        W_item.T,
        W_cat.T,
    )
    return outT.T
